# Initial kernel scaffold; baseline (speedup 1.0000x reference)
import jax
import jax.numpy as jnp
from jax.experimental import pallas as pl


def kernel(userID, itemID, L_row, L_col, L_data, user_emb, item_emb, gW1, gb1, gW2, gb2, W1, b1, W2, b2, W3, b3):
    raise NotImplementedError("write your pallas kernel here")


# speedup vs baseline: 5.5447x; 5.5447x over previous
import functools

import jax
import jax.numpy as jnp
from jax import lax
from jax.experimental import pallas as pl
from jax.experimental.pallas import tpu as pltpu
from jax.experimental.pallas import tpu_sc as plsc

NC = 2
NS = 16
LANES = 16


def _mesh():
  return plsc.VectorSubcoreMesh(core_axis_name="c", subcore_axis_name="s")


_SC_PARAMS = pltpu.CompilerParams(use_tc_tiling_on_sc=False)


def _build_inv_kernel(n_pad, n_slot_rows):
  chunk = n_pad // NS

  @functools.partial(
      pl.kernel,
      out_type=jax.ShapeDtypeStruct((n_pad,), jnp.int32),
      mesh=_mesh(),
      compiler_params=_SC_PARAMS,
      scratch_types=[
          pltpu.VMEM((chunk,), jnp.int32),
          pltpu.VMEM((n_slot_rows // NS, 128), jnp.int32),
          pltpu.VMEM((n_slot_rows // NS, 128), jnp.int32),
      ],
  )
  def k(nodes_hbm, slots_hbm, inv_hbm, initbuf, nodes_v, slots_v):
    cid = lax.axis_index("c")
    sid = lax.axis_index("s")
    jrows = n_slot_rows // NS

    @pl.when(cid == 0)
    def _():
      def fill(i, _):
        initbuf[pl.ds(i * LANES, LANES)] = jnp.full((LANES,), -1, jnp.int32)
        return 0
      lax.fori_loop(0, chunk // LANES, fill, 0)
      pltpu.sync_copy(initbuf, inv_hbm.at[pl.ds(sid * chunk, chunk)])
      plsc.subcore_barrier()
      pltpu.sync_copy(nodes_hbm.at[pl.ds(sid * jrows, jrows)], nodes_v)
      pltpu.sync_copy(slots_hbm.at[pl.ds(sid * jrows, jrows)], slots_v)
      for j in range(jrows):
        pltpu.sync_copy(slots_v.at[j], inv_hbm.at[nodes_v.at[j]])

  return k


def _edge_accum_kernel(erows, bpt, acc_rows, dump_row, n_pad, d):
  C = 256
  JR = C // 128
  rpt = acc_rows // NS

  @functools.partial(
      pl.kernel,
      out_type=jax.ShapeDtypeStruct((2 * acc_rows, 2 * d), jnp.float32),
      mesh=_mesh(),
      compiler_params=_SC_PARAMS,
      scratch_types=[
          pltpu.VMEM_SHARED((acc_rows, 2 * d), jnp.float32),
          pltpu.VMEM((JR, 128), jnp.int32),
          pltpu.VMEM((JR, 128), jnp.int32),
          pltpu.VMEM((JR, 128), jnp.float32),
          pltpu.VMEM((JR, 128), jnp.int32),
          pltpu.VMEM((JR, 128), jnp.int32),
          pltpu.VMEM((C, d), jnp.float32),
          pltpu.VMEM((C, 2 * d), jnp.float32),
          pltpu.VMEM((64, 2 * d), jnp.float32),
          pltpu.SemaphoreType.DMA,
      ],
  )
  def k(rows_hbm, cols_hbm, vals_hbm, inv_hbm, f_hbm, p_hbm,
        acc, rows_v, cols_v, vals_v, s_v, s2_v, g_v, contrib, zbuf, sem):
    cid = lax.axis_index("c")
    sid = lax.axis_index("s")
    wid = cid * NS + sid

    def zfill(i, _):
      r = i // (2 * d // LANES)
      c16 = (i % (2 * d // LANES)) * LANES
      zbuf[r, pl.ds(c16, LANES)] = jnp.zeros((LANES,), jnp.float32)
      return 0
    lax.fori_loop(0, 64 * (2 * d // LANES), zfill, 0)
    full64 = rpt // 64
    for z in range(full64):
      pltpu.sync_copy(zbuf, acc.at[pl.ds(sid * rpt + z * 64, 64)])
    rem = rpt - full64 * 64
    if rem:
      pltpu.sync_copy(zbuf.at[pl.ds(0, rem)],
                      acc.at[pl.ds(sid * rpt + full64 * 64, rem)])
    plsc.subcore_barrier()

    def block(b, _):
      blk = (wid * bpt + b) * JR
      pltpu.sync_copy(rows_hbm.at[pl.ds(blk, JR)], rows_v)
      pltpu.sync_copy(cols_hbm.at[pl.ds(blk, JR)], cols_v)
      pltpu.sync_copy(vals_hbm.at[pl.ds(blk, JR)], vals_v)
      for j in range(JR):
        pltpu.async_copy(inv_hbm.at[rows_v.at[j]], s_v.at[j], sem).wait()
        pltpu.async_copy(f_hbm.at[cols_v.at[j]],
                         g_v.at[pl.ds(j * 128, 128)], sem).wait()
      for j in range(JR):
        for jj in range(128 // LANES):
          sl = s_v[j, pl.ds(jj * LANES, LANES)]
          s2_v[j, pl.ds(jj * LANES, LANES)] = jnp.where(
              sl < 0, jnp.int32(dump_row), sl)

      def grp(gi, _):
        vv = vals_v[gi // 8, pl.ds((gi % 8) * LANES, LANES)]
        for l in range(LANES):
          val = vv[l]
          e = gi * LANES + l
          for g16 in range(d // LANES):
            gk = g_v[e, pl.ds(g16 * LANES, LANES)]
            a = val * gk
            contrib[e, pl.ds(g16 * LANES, LANES)] = a
            contrib[e, pl.ds(d + g16 * LANES, LANES)] = a * gk
        return 0
      lax.fori_loop(0, C // LANES, grp, 0)

      for j in range(JR):
        pltpu.sync_copy(contrib.at[pl.ds(j * 128, 128)],
                        acc.at[s2_v.at[j]], add=True)
      return 0
    lax.fori_loop(0, bpt, block, 0)

    plsc.subcore_barrier()
    pltpu.sync_copy(acc.at[pl.ds(sid * rpt, rpt)],
                    p_hbm.at[pl.ds(cid * acc_rows + sid * rpt, rpt)])

  return k


def _slot_gather_kernel(nslots, acc_rows, n_pad, d):
  spw = nslots // (NC * NS)
  JR = spw // 128

  @functools.partial(
      pl.kernel,
      out_type=(jax.ShapeDtypeStruct((nslots, 2 * d), jnp.float32),
                jax.ShapeDtypeStruct((nslots, 2 * d), jnp.float32),
                jax.ShapeDtypeStruct((nslots, d), jnp.float32)),
      mesh=_mesh(),
      compiler_params=_SC_PARAMS,
      scratch_types=[
          pltpu.VMEM((JR, 128), jnp.int32),
          pltpu.VMEM((JR, 128), jnp.int32),
          pltpu.VMEM((JR, 128), jnp.int32),
          pltpu.VMEM((spw, 2 * d), jnp.float32),
          pltpu.VMEM((spw, 2 * d), jnp.float32),
          pltpu.VMEM((spw, d), jnp.float32),
          pltpu.SemaphoreType.DMA,
      ],
  )
  def k(nodes_hbm, inv_hbm, p_hbm, f_hbm, a0_hbm, a1_hbm, fb_hbm,
        nodes_v, canon_v, canon2_v, a0_v, a1_v, fb_v, sem):
    cid = lax.axis_index("c")
    sid = lax.axis_index("s")
    wid = cid * NS + sid
    pltpu.sync_copy(nodes_hbm.at[pl.ds(wid * JR, JR)], nodes_v)
    for j in range(JR):
      pltpu.async_copy(inv_hbm.at[nodes_v.at[j]], canon_v.at[j], sem).wait()
      for jj in range(128 // LANES):
        canon2_v[j, pl.ds(jj * LANES, LANES)] = (
            canon_v[j, pl.ds(jj * LANES, LANES)] + jnp.int32(acc_rows))
      pltpu.async_copy(p_hbm.at[canon_v.at[j]],
                       a0_v.at[pl.ds(j * 128, 128)], sem).wait()
      pltpu.async_copy(p_hbm.at[canon2_v.at[j]],
                       a1_v.at[pl.ds(j * 128, 128)], sem).wait()
      pltpu.async_copy(f_hbm.at[nodes_v.at[j]],
                       fb_v.at[pl.ds(j * 128, 128)], sem).wait()
    pltpu.sync_copy(a0_v, a0_hbm.at[pl.ds(wid * spw, spw)])
    pltpu.sync_copy(a1_v, a1_hbm.at[pl.ds(wid * spw, spw)])
    pltpu.sync_copy(fb_v, fb_hbm.at[pl.ds(wid * spw, spw)])

  return k


def _dense_tc(a0g, a1g, fb, gW1, gb1, gW2, gb2, W1, b1, W2, b2, W3, b3):
  nslots, d2 = a0g.shape
  d = d2 // 2
  bsz = nslots // 2

  def body(a0_r, a1_r, fb_r, gW1_r, gb1_r, gW2_r, gb2_r,
           W1_r, b1_r, W2_r, b2_r, W3_r, b3_r, out_r):
    asum = a0_r[...] + a1_r[...]
    agg = asum[:, :d]
    agg2 = asum[:, d:]
    f = fb_r[...]
    hp = jax.lax.Precision.HIGHEST
    inter1 = jnp.dot(agg + f, gW1_r[...].T, precision=hp) + gb1_r[...]
    inter2 = jnp.dot(agg2, gW2_r[...].T, precision=hp) + gb2_r[...]
    x = inter1 + inter2
    feats = jnp.where(x >= 0, x, 0.01 * x)
    embed = jnp.concatenate(
        [f[:bsz], feats[:bsz], f[bsz:], feats[bsz:]], axis=1)
    h = jnp.dot(embed, W1_r[...].T, precision=hp) + b1_r[...]
    h = jnp.maximum(h, 0.0)
    h = jnp.dot(h, W2_r[...].T, precision=hp) + b2_r[...]
    h = jnp.maximum(h, 0.0)
    out_r[...] = jnp.dot(h, W3_r[...].T, precision=hp) + b3_r[...]

  w3p = jnp.zeros((128, W3.shape[1]), W3.dtype).at[0].set(W3[0])
  b3p = jnp.broadcast_to(b3.reshape(1, 1), (1, 128))
  return pl.pallas_call(
      body,
      out_shape=jax.ShapeDtypeStruct((bsz, 128), jnp.float32),
  )(a0g, a1g, fb, gW1, gb1.reshape(1, -1), gW2, gb2.reshape(1, -1),
    W1, b1.reshape(1, -1), W2, b2.reshape(1, -1), w3p, b3p)


def kernel(userID, itemID, L_row, L_col, L_data, user_emb, item_emb,
           gW1, gb1, gW2, gb2, W1, b1, W2, b2, W3, b3):
  U, d = user_emb.shape
  I = item_emb.shape[0]
  N = U + I
  B = userID.shape[0]
  E = L_row.shape[0]
  nslots = 2 * B

  feats_tab = jnp.concatenate([user_emb, item_emb], axis=0)
  nodes = jnp.concatenate([userID, itemID + U]).astype(jnp.int32)
  nodes2d = nodes.reshape(nslots // 128, 128)
  slots2d = jnp.arange(nslots, dtype=jnp.int32).reshape(nslots // 128, 128)

  chunk = ((N + NS - 1) // NS + 7) // 8 * 8
  n_pad = chunk * NS

  inv = _build_inv_kernel(n_pad, nslots // 128)(nodes2d, slots2d)

  C = 256
  NW = NC * NS
  bpt = -(-E // (NW * C))
  e_pad = bpt * NW * C
  pad = e_pad - E
  rows2d = jnp.concatenate(
      [L_row, jnp.zeros((pad,), jnp.int32)]).reshape(e_pad // 128, 128)
  cols2d = jnp.concatenate(
      [L_col, jnp.zeros((pad,), jnp.int32)]).reshape(e_pad // 128, 128)
  vals2d = jnp.concatenate(
      [L_data, jnp.zeros((pad,), jnp.float32)]).reshape(e_pad // 128, 128)

  acc_rows = ((nslots + 1 + NS * 8 - 1) // (NS * 8)) * NS * 8
  dump_row = nslots

  p = _edge_accum_kernel(e_pad // 128, bpt, acc_rows, dump_row,
                         n_pad, d)(rows2d, cols2d, vals2d, inv, feats_tab)
  a0g, a1g, fb = _slot_gather_kernel(nslots, acc_rows, n_pad, d)(
      nodes2d, inv, p, feats_tab)

  out = _dense_tc(a0g, a1g, fb, gW1, gb1, gW2, gb2, W1, b1, W2, b2, W3, b3)
  return out[:, 0]


# speedup vs baseline: 9.0999x; 1.6412x over previous
import functools

import jax
import jax.numpy as jnp
from jax import lax
from jax.experimental import pallas as pl
from jax.experimental.pallas import tpu as pltpu
from jax.experimental.pallas import tpu_sc as plsc

NC = 2
NS = 16
LANES = 16


def _mesh():
  return plsc.VectorSubcoreMesh(core_axis_name="c", subcore_axis_name="s")


_SC_PARAMS = pltpu.CompilerParams(use_tc_tiling_on_sc=False)
_SC_PARAMS_NL = pltpu.CompilerParams(use_tc_tiling_on_sc=False,
                                     needs_layout_passes=False)


def _build_inv_kernel(n_pad, n_slot_rows):
  chunk = n_pad // NS

  @functools.partial(
      pl.kernel,
      out_type=jax.ShapeDtypeStruct((n_pad,), jnp.int32),
      mesh=_mesh(),
      compiler_params=_SC_PARAMS,
      scratch_types=[
          pltpu.VMEM((chunk,), jnp.int32),
          pltpu.VMEM((n_slot_rows // NS, 128), jnp.int32),
          pltpu.VMEM((n_slot_rows // NS, 128), jnp.int32),
      ],
  )
  def k(nodes_hbm, slots_hbm, inv_hbm, initbuf, nodes_v, slots_v):
    cid = lax.axis_index("c")
    sid = lax.axis_index("s")
    jrows = n_slot_rows // NS

    @pl.when(cid == 0)
    def _():
      def fill(i, _):
        initbuf[pl.ds(i * LANES, LANES)] = jnp.full((LANES,), -1, jnp.int32)
        return 0
      lax.fori_loop(0, chunk // LANES, fill, 0)
      pltpu.sync_copy(initbuf, inv_hbm.at[pl.ds(sid * chunk, chunk)])
      plsc.subcore_barrier()
      pltpu.sync_copy(nodes_hbm.at[pl.ds(sid * jrows, jrows)], nodes_v)
      pltpu.sync_copy(slots_hbm.at[pl.ds(sid * jrows, jrows)], slots_v)
      for j in range(jrows):
        pltpu.sync_copy(slots_v.at[j], inv_hbm.at[nodes_v.at[j]])

  return k


def _edge_accum_kernel(erows, bpt, acc_rows, dump_row, n_pad, d):
  C = 1024
  JR = C // 128
  rpt = acc_rows // NS

  @functools.partial(
      pl.kernel,
      out_type=jax.ShapeDtypeStruct((2 * acc_rows, 2 * d), jnp.float32),
      mesh=_mesh(),
      compiler_params=_SC_PARAMS_NL,
      scratch_types=[
          pltpu.VMEM_SHARED((acc_rows, 2 * d), jnp.float32),
          pltpu.VMEM((JR, 128), jnp.int32),
          pltpu.VMEM((JR, 128), jnp.int32),
          pltpu.VMEM((JR, 128), jnp.float32),
          pltpu.VMEM((JR, 128), jnp.int32),
          pltpu.VMEM((C + 128,), jnp.int32),
          pltpu.VMEM((C + 128,), jnp.int32),
          pltpu.VMEM((C + 128,), jnp.float32),
          pltpu.VMEM((1, 128), jnp.int32),
          pltpu.VMEM((128, d), jnp.float32),
          pltpu.VMEM((128, 2 * d), jnp.float32),
          pltpu.VMEM((64, 2 * d), jnp.float32),
          pltpu.SemaphoreType.DMA,
          pltpu.SemaphoreType.DMA,
      ],
  )
  def k(rows_hbm, cols_hbm, vals_hbm, inv_hbm, f_hbm, p_hbm,
        acc, rows_v, cols_v, vals_v, s_v, cs_buf, cc_buf, cv_buf, s2d,
        g_v, contrib, zbuf, sem, sem2):
    cid = lax.axis_index("c")
    sid = lax.axis_index("s")
    wid = cid * NS + sid

    def zfill(i, _):
      r = i // (2 * d // LANES)
      c16 = (i % (2 * d // LANES)) * LANES
      zbuf[r, pl.ds(c16, LANES)] = jnp.zeros((LANES,), jnp.float32)
      return 0
    lax.fori_loop(0, 64 * (2 * d // LANES), zfill, 0)
    full64 = rpt // 64
    for z in range(full64):
      pltpu.sync_copy(zbuf, acc.at[pl.ds(sid * rpt + z * 64, 64)])
    rem = rpt - full64 * 64
    if rem:
      pltpu.sync_copy(zbuf.at[pl.ds(0, rem)],
                      acc.at[pl.ds(sid * rpt + full64 * 64, rem)])
    plsc.subcore_barrier()

    def block(b, _):
      blk = (wid * bpt + b) * JR
      pltpu.sync_copy(rows_hbm.at[pl.ds(blk, JR)], rows_v)
      pltpu.sync_copy(cols_hbm.at[pl.ds(blk, JR)], cols_v)
      pltpu.sync_copy(vals_hbm.at[pl.ds(blk, JR)], vals_v)
      cps = [pltpu.async_copy(inv_hbm.at[rows_v.at[j]], s_v.at[j], sem)
             for j in range(JR)]
      for cp in cps:
        cp.wait()

      def cgrp(gi, cnt):
        j = gi // 8
        off = (gi % 8) * LANES
        sl = s_v[j, pl.ds(off, LANES)]
        cl = cols_v[j, pl.ds(off, LANES)]
        vl = vals_v[j, pl.ds(off, LANES)]
        m = sl >= 0
        ss, cc = plsc.sort_key_val(sl, cl, descending=True)
        ss2, vv2 = plsc.sort_key_val(sl, vl, descending=True)
        cs_buf[pl.ds(cnt, LANES)] = ss
        cc_buf[pl.ds(cnt, LANES)] = cc
        cv_buf[pl.ds(cnt, LANES)] = vv2
        return cnt + plsc.all_reduce_population_count(m)[0]
      cnt = lax.fori_loop(0, C // LANES, cgrp, jnp.int32(0))

      zf = jnp.zeros((LANES,), jnp.float32)
      zdump = jnp.full((LANES,), dump_row, jnp.int32)
      zcol = jnp.zeros((LANES,), jnp.int32)
      for t in range(8):
        cs_buf[pl.ds(cnt + t * LANES, LANES)] = zdump
        cc_buf[pl.ds(cnt + t * LANES, LANES)] = zcol
        cv_buf[pl.ds(cnt + t * LANES, LANES)] = zf

      def sub(j2, _):
        base = j2 * 128
        pltpu.async_copy(f_hbm.at[cc_buf.at[pl.ds(base, 128)]], g_v,
                         sem2).wait()
        for t in range(8):
          s2d[0, pl.ds(t * LANES, LANES)] = cs_buf[pl.ds(base + t * LANES,
                                                         LANES)]
        def grp(gi, _):
          vv = cv_buf[pl.ds(base + gi * LANES, LANES)]
          for l in range(LANES):
            val = vv[l]
            e = gi * LANES + l
            for g16 in range(d // LANES):
              gk = g_v[e, pl.ds(g16 * LANES, LANES)]
              a = val * gk
              contrib[e, pl.ds(g16 * LANES, LANES)] = a
              contrib[e, pl.ds(d + g16 * LANES, LANES)] = a * gk
          return 0
        lax.fori_loop(0, 128 // LANES, grp, 0)
        pltpu.sync_copy(contrib, acc.at[s2d.at[0]], add=True)
        return 0
      nsub = (cnt + 127) // 128
      lax.fori_loop(0, nsub, sub, 0)
      return 0
    lax.fori_loop(0, bpt, block, 0)

    plsc.subcore_barrier()
    pltpu.sync_copy(acc.at[pl.ds(sid * rpt, rpt)],
                    p_hbm.at[pl.ds(cid * acc_rows + sid * rpt, rpt)])

  return k


def _slot_gather_kernel(nslots, acc_rows, n_pad, d):
  spw = nslots // (NC * NS)
  JR = spw // 128

  @functools.partial(
      pl.kernel,
      out_type=(jax.ShapeDtypeStruct((nslots, 2 * d), jnp.float32),
                jax.ShapeDtypeStruct((nslots, 2 * d), jnp.float32),
                jax.ShapeDtypeStruct((nslots, d), jnp.float32)),
      mesh=_mesh(),
      compiler_params=_SC_PARAMS,
      scratch_types=[
          pltpu.VMEM((JR, 128), jnp.int32),
          pltpu.VMEM((JR, 128), jnp.int32),
          pltpu.VMEM((JR, 128), jnp.int32),
          pltpu.VMEM((spw, 2 * d), jnp.float32),
          pltpu.VMEM((spw, 2 * d), jnp.float32),
          pltpu.VMEM((spw, d), jnp.float32),
          pltpu.SemaphoreType.DMA,
      ],
  )
  def k(nodes_hbm, inv_hbm, p_hbm, f_hbm, a0_hbm, a1_hbm, fb_hbm,
        nodes_v, canon_v, canon2_v, a0_v, a1_v, fb_v, sem):
    cid = lax.axis_index("c")
    sid = lax.axis_index("s")
    wid = cid * NS + sid
    pltpu.sync_copy(nodes_hbm.at[pl.ds(wid * JR, JR)], nodes_v)
    for j in range(JR):
      pltpu.async_copy(inv_hbm.at[nodes_v.at[j]], canon_v.at[j], sem).wait()
      for jj in range(128 // LANES):
        canon2_v[j, pl.ds(jj * LANES, LANES)] = (
            canon_v[j, pl.ds(jj * LANES, LANES)] + jnp.int32(acc_rows))
      pltpu.async_copy(p_hbm.at[canon_v.at[j]],
                       a0_v.at[pl.ds(j * 128, 128)], sem).wait()
      pltpu.async_copy(p_hbm.at[canon2_v.at[j]],
                       a1_v.at[pl.ds(j * 128, 128)], sem).wait()
      pltpu.async_copy(f_hbm.at[nodes_v.at[j]],
                       fb_v.at[pl.ds(j * 128, 128)], sem).wait()
    pltpu.sync_copy(a0_v, a0_hbm.at[pl.ds(wid * spw, spw)])
    pltpu.sync_copy(a1_v, a1_hbm.at[pl.ds(wid * spw, spw)])
    pltpu.sync_copy(fb_v, fb_hbm.at[pl.ds(wid * spw, spw)])

  return k


def _dense_tc(a0g, a1g, fb, gW1, gb1, gW2, gb2, W1, b1, W2, b2, W3, b3):
  nslots, d2 = a0g.shape
  d = d2 // 2
  bsz = nslots // 2

  def body(a0_r, a1_r, fb_r, gW1_r, gb1_r, gW2_r, gb2_r,
           W1_r, b1_r, W2_r, b2_r, W3_r, b3_r, out_r):
    asum = a0_r[...] + a1_r[...]
    agg = asum[:, :d]
    agg2 = asum[:, d:]
    f = fb_r[...]
    hp = jax.lax.Precision.HIGHEST
    inter1 = jnp.dot(agg + f, gW1_r[...].T, precision=hp) + gb1_r[...]
    inter2 = jnp.dot(agg2, gW2_r[...].T, precision=hp) + gb2_r[...]
    x = inter1 + inter2
    feats = jnp.where(x >= 0, x, 0.01 * x)
    embed = jnp.concatenate(
        [f[:bsz], feats[:bsz], f[bsz:], feats[bsz:]], axis=1)
    h = jnp.dot(embed, W1_r[...].T, precision=hp) + b1_r[...]
    h = jnp.maximum(h, 0.0)
    h = jnp.dot(h, W2_r[...].T, precision=hp) + b2_r[...]
    h = jnp.maximum(h, 0.0)
    out_r[...] = jnp.dot(h, W3_r[...].T, precision=hp) + b3_r[...]

  w3p = jnp.zeros((128, W3.shape[1]), W3.dtype).at[0].set(W3[0])
  b3p = jnp.broadcast_to(b3.reshape(1, 1), (1, 128))
  return pl.pallas_call(
      body,
      out_shape=jax.ShapeDtypeStruct((bsz, 128), jnp.float32),
  )(a0g, a1g, fb, gW1, gb1.reshape(1, -1), gW2, gb2.reshape(1, -1),
    W1, b1.reshape(1, -1), W2, b2.reshape(1, -1), w3p, b3p)


def kernel(userID, itemID, L_row, L_col, L_data, user_emb, item_emb,
           gW1, gb1, gW2, gb2, W1, b1, W2, b2, W3, b3):
  U, d = user_emb.shape
  I = item_emb.shape[0]
  N = U + I
  B = userID.shape[0]
  E = L_row.shape[0]
  nslots = 2 * B

  feats_tab = jnp.concatenate([user_emb, item_emb], axis=0)
  nodes = jnp.concatenate([userID, itemID + U]).astype(jnp.int32)
  nodes2d = nodes.reshape(nslots // 128, 128)
  slots2d = jnp.arange(nslots, dtype=jnp.int32).reshape(nslots // 128, 128)

  chunk = ((N + NS - 1) // NS + 7) // 8 * 8
  n_pad = chunk * NS

  inv = _build_inv_kernel(n_pad, nslots // 128)(nodes2d, slots2d)

  C = 1024
  NW = NC * NS
  bpt = -(-E // (NW * C))
  e_pad = bpt * NW * C
  pad = e_pad - E
  rows2d = jnp.concatenate(
      [L_row, jnp.zeros((pad,), jnp.int32)]).reshape(e_pad // 128, 128)
  cols2d = jnp.concatenate(
      [L_col, jnp.zeros((pad,), jnp.int32)]).reshape(e_pad // 128, 128)
  vals2d = jnp.concatenate(
      [L_data, jnp.zeros((pad,), jnp.float32)]).reshape(e_pad // 128, 128)

  acc_rows = ((nslots + 1 + NS * 8 - 1) // (NS * 8)) * NS * 8
  dump_row = nslots

  p = _edge_accum_kernel(e_pad // 128, bpt, acc_rows, dump_row,
                         n_pad, d)(rows2d, cols2d, vals2d, inv, feats_tab)
  a0g, a1g, fb = _slot_gather_kernel(nslots, acc_rows, n_pad, d)(
      nodes2d, inv, p, feats_tab)

  out = _dense_tc(a0g, a1g, fb, gW1, gb1, gW2, gb2, W1, b1, W2, b2, W3, b3)
  return out[:, 0]
